# speedup vs baseline: 2.2964x; 2.2964x over previous
import jax
import jax.numpy as jnp
from jax.experimental import pallas as pl


def kernel(x, table):
    bsz, seq_len = x.shape
    ctx, dim = table.shape
    tbl = table[:seq_len]

    SBLK = 512

    def body(t_ref, o_ref):
        o_ref[...] = jnp.broadcast_to(t_ref[...][None], (bsz, SBLK, dim))

    out = pl.pallas_call(
        body,
        grid=(seq_len // SBLK,),
        in_specs=[pl.BlockSpec((SBLK, dim), lambda i: (i, 0))],
        out_specs=pl.BlockSpec((bsz, SBLK, dim), lambda i: (0, i, 0)),
        out_shape=jax.ShapeDtypeStruct((bsz, seq_len, dim), table.dtype),
    )(tbl)
    return out
